# Initial kernel scaffold; baseline (speedup 1.0000x reference)
import jax
import jax.numpy as jnp
from jax.experimental import pallas as pl


def kernel(x, edge_index, adj_values, W_in, b_in, rel_W, W_self, b_self, W_out, b_out):
    raise NotImplementedError("write your pallas kernel here")


# speedup vs baseline: 2.2689x; 2.2689x over previous
import functools

import jax
import jax.numpy as jnp
from jax import lax
from jax.experimental import pallas as pl
from jax.experimental.pallas import tpu as pltpu
from jax.experimental.pallas import tpu_sc as plsc

N = 10000
D = 128
R = 8
E = 40000

NC = 2
NS = 16
NW = NC * NS
EPT = (R * E) // NW
CH = 80
NCHUNK = EPT // CH
RPT = 624
ZR = 208
BT = 1000

_F32 = jnp.float32
_DOT = functools.partial(
    lax.dot_general,
    dimension_numbers=(((1,), (1,)), ((), ())),
    preferred_element_type=_F32,
    precision=lax.Precision.HIGHEST,
)


def _tables_body(x_ref, w_in_ref, b_in_ref, rel_w_ref, w_self_ref, b_self_ref,
                 out0_ref, table_ref):
    h = jnp.maximum(_DOT(x_ref[...], w_in_ref[...]) + b_in_ref[...], 0.0)
    out0_ref[...] = _DOT(h, w_self_ref[...]) + b_self_ref[...]
    for r in range(R):
        table_ref[r] = _DOT(h, rel_w_ref[r])


def _tables_call(x, W_in, b_in, rel_W, W_self, b_self):
    return pl.pallas_call(
        _tables_body,
        grid=(N // BT,),
        in_specs=[
            pl.BlockSpec((BT, D), lambda i: (i, 0)),
            pl.BlockSpec((D, D), lambda i: (0, 0)),
            pl.BlockSpec((1, D), lambda i: (0, 0)),
            pl.BlockSpec((R, D, D), lambda i: (0, 0, 0)),
            pl.BlockSpec((D, D), lambda i: (0, 0)),
            pl.BlockSpec((1, D), lambda i: (0, 0)),
        ],
        out_specs=[
            pl.BlockSpec((BT, D), lambda i: (i, 0)),
            pl.BlockSpec((R, BT, D), lambda i: (0, i, 0)),
        ],
        out_shape=[
            jax.ShapeDtypeStruct((N, D), _F32),
            jax.ShapeDtypeStruct((R, N, D), _F32),
        ],
    )(x, W_in, b_in, rel_W, W_self, b_self)


_MESH = plsc.VectorSubcoreMesh(core_axis_name="c", subcore_axis_name="s")


@functools.partial(
    pl.kernel,
    out_type=jax.ShapeDtypeStruct((NC, N, D), _F32),
    mesh=_MESH,
    scratch_types=[
        pltpu.VMEM_SHARED((N, D), _F32),
        pltpu.VMEM((CH, D), _F32),
        pltpu.VMEM((CH,), jnp.int32),
        pltpu.VMEM((CH,), jnp.int32),
        pltpu.VMEM((CH,), _F32),
        pltpu.VMEM((ZR, D), _F32),
        pltpu.SemaphoreType.DMA,
    ],
    compiler_params=pltpu.CompilerParams(needs_layout_passes=False),
)
def _sc_agg(table_hbm, src_hbm, dst_hbm, val_hbm, out_hbm,
            acc, rows_v, src_v, dst_v, val_v, zbuf, sem):
    cid = lax.axis_index("c")
    sid = lax.axis_index("s")
    wid = cid * NS + sid
    row0 = sid * RPT

    zv = jnp.zeros((16,), _F32)

    def _zrow(i, carry):
        for j in range(D // 16):
            zbuf[i, pl.ds(j * 16, 16)] = zv
        return carry

    lax.fori_loop(0, ZR, _zrow, 0)
    for kk in range(RPT // ZR):
        pltpu.sync_copy(zbuf, acc.at[pl.ds(row0 + kk * ZR, ZR)])
    @pl.when(sid == NS - 1)
    def _ztail():
        pltpu.sync_copy(zbuf.at[pl.ds(0, N - NS * RPT)],
                        acc.at[pl.ds(NS * RPT, N - NS * RPT)])
    plsc.subcore_barrier()

    ebase = wid * EPT

    def _chunk(k, carry):
        off = ebase + k * CH
        pltpu.sync_copy(src_hbm.at[pl.ds(off, CH)], src_v)
        pltpu.sync_copy(dst_hbm.at[pl.ds(off, CH)], dst_v)
        pltpu.sync_copy(val_hbm.at[pl.ds(off, CH)], val_v)
        pltpu.async_copy(table_hbm.at[src_v], rows_v, sem).wait()

        def _scale(e, c2):
            sv = plsc.load_gather(val_v, [jnp.full((16,), e, jnp.int32)])
            for j in range(D // 16):
                rows_v[e, pl.ds(j * 16, 16)] = rows_v[e, pl.ds(j * 16, 16)] * sv
            return c2

        lax.fori_loop(0, CH, _scale, 0)
        pltpu.sync_copy(rows_v, acc.at[dst_v], add=True)
        return carry

    lax.fori_loop(0, NCHUNK, _chunk, 0)
    plsc.subcore_barrier()

    pltpu.sync_copy(acc.at[pl.ds(row0, RPT)], out_hbm.at[cid, pl.ds(row0, RPT)])
    @pl.when(sid == NS - 1)
    def _wtail():
        pltpu.sync_copy(acc.at[pl.ds(NS * RPT, N - NS * RPT)],
                        out_hbm.at[cid, pl.ds(NS * RPT, N - NS * RPT)])


def _final_body(out0_ref, part_ref, w_out_ref, b_out_ref, y_ref):
    z = jnp.maximum(out0_ref[...] + part_ref[0] + part_ref[1], 0.0)
    y_ref[...] = _DOT(z, w_out_ref[...]) + b_out_ref[...]


def _final_call(out0, partials, W_out, b_out):
    return pl.pallas_call(
        _final_body,
        grid=(N // BT,),
        in_specs=[
            pl.BlockSpec((BT, D), lambda i: (i, 0)),
            pl.BlockSpec((NC, BT, D), lambda i: (0, i, 0)),
            pl.BlockSpec((D, D), lambda i: (0, 0)),
            pl.BlockSpec((1, D), lambda i: (0, 0)),
        ],
        out_specs=pl.BlockSpec((BT, D), lambda i: (i, 0)),
        out_shape=jax.ShapeDtypeStruct((N, D), _F32),
    )(out0, partials, W_out, b_out)


def kernel(x, edge_index, adj_values, W_in, b_in, rel_W, W_self, b_self,
           W_out, b_out):
    src = (edge_index[:, 0, :]
           + (jnp.arange(R, dtype=jnp.int32) * N)[:, None]).reshape(-1)
    dst = edge_index[:, 1, :].reshape(-1)
    val = adj_values.reshape(-1)
    out0, table = _tables_call(x, W_in, b_in.reshape(1, D), rel_W,
                               W_self, b_self.reshape(1, D))
    partials = _sc_agg(table.reshape(R * N, D), src, dst, val)
    return _final_call(out0, partials, W_out, b_out.reshape(1, D))


# speedup vs baseline: 4.5724x; 2.0153x over previous
import functools

import jax
import jax.numpy as jnp
from jax import lax
from jax.experimental import pallas as pl
from jax.experimental.pallas import tpu as pltpu
from jax.experimental.pallas import tpu_sc as plsc

N = 10000
D = 128
R = 8
E = 40000

NC = 2
NS = 16
NW = NC * NS
EPT = (R * E) // NW
CH = 80
NCHUNK = EPT // CH
LASTC = NCHUNK - 1
RPT = 624
ZR = 208
BT = 1000

_F32 = jnp.float32
_DOT_T = functools.partial(
    lax.dot_general,
    dimension_numbers=(((1,), (1,)), ((), ())),
    preferred_element_type=_F32,
    precision=lax.Precision.HIGHEST,
)
_DOT = functools.partial(
    lax.dot_general,
    dimension_numbers=(((1,), (0,)), ((), ())),
    preferred_element_type=_F32,
    precision=lax.Precision.HIGHEST,
)


def _tables_body(x_ref, w_in_ref, b_in_ref, w_big_ref, b_self_ref,
                 out0_ref, table_ref):
    h = jnp.maximum(_DOT_T(x_ref[...], w_in_ref[...]) + b_in_ref[...], 0.0)
    big = _DOT(h, w_big_ref[...])
    out0_ref[...] = big[:, :D] + b_self_ref[...]
    for r in range(R):
        table_ref[r] = big[:, (r + 1) * D:(r + 2) * D]


def _tables_call(x, W_in, b_in, W_big, b_self):
    return pl.pallas_call(
        _tables_body,
        grid=(N // BT,),
        in_specs=[
            pl.BlockSpec((BT, D), lambda i: (i, 0)),
            pl.BlockSpec((D, D), lambda i: (0, 0)),
            pl.BlockSpec((1, D), lambda i: (0, 0)),
            pl.BlockSpec((D, (R + 1) * D), lambda i: (0, 0)),
            pl.BlockSpec((1, D), lambda i: (0, 0)),
        ],
        out_specs=[
            pl.BlockSpec((BT, D), lambda i: (i, 0)),
            pl.BlockSpec((R, BT, D), lambda i: (0, i, 0)),
        ],
        out_shape=[
            jax.ShapeDtypeStruct((N, D), _F32),
            jax.ShapeDtypeStruct((R, N, D), _F32),
        ],
    )(x, W_in, b_in, W_big, b_self)


_MESH = plsc.VectorSubcoreMesh(core_axis_name="c", subcore_axis_name="s")


@functools.partial(
    pl.kernel,
    out_type=jax.ShapeDtypeStruct((NC, N, D), _F32),
    mesh=_MESH,
    scratch_types=[
        pltpu.VMEM_SHARED((N, D), _F32),
        pltpu.VMEM((CH, D), _F32),
        pltpu.VMEM((CH, D), _F32),
        pltpu.VMEM((3, CH), jnp.int32),
        pltpu.VMEM((3, CH), jnp.int32),
        pltpu.VMEM((CH,), jnp.int32),
        pltpu.VMEM((CH,), jnp.int32),
        pltpu.VMEM((CH,), _F32),
        pltpu.VMEM((CH,), _F32),
        pltpu.VMEM((ZR, D), _F32),
        pltpu.SemaphoreType.DMA,
        pltpu.SemaphoreType.DMA,
        pltpu.SemaphoreType.DMA,
        pltpu.SemaphoreType.DMA,
        pltpu.SemaphoreType.DMA,
    ],
    compiler_params=pltpu.CompilerParams(needs_layout_passes=False),
)
def _sc_agg(table_hbm, pk_hbm, out_hbm,
            acc, rows0, rows1, pk0, pk1, sidx0, sidx1, sval0, sval1, zbuf,
            gsem, psem0, psem1, ssem0, ssem1):
    cid = lax.axis_index("c")
    sid = lax.axis_index("s")
    wid = cid * NS + sid
    row0 = sid * RPT

    pkb = (pk0, pk1)
    rowsb = (rows0, rows1)
    sidxb = (sidx0, sidx1)
    svalb = (sval0, sval1)
    psemb = (psem0, psem1)
    ssemb = (ssem0, ssem1)

    pltpu.async_copy(pk_hbm.at[wid, 0], pk0, psem0)
    pltpu.async_copy(pk_hbm.at[wid, 1], pk1, psem1)

    zv = jnp.zeros((16,), _F32)

    def _zrow(i, carry):
        for j in range(D // 16):
            zbuf[i, pl.ds(j * 16, 16)] = zv
        return carry

    lax.fori_loop(0, ZR, _zrow, 0)
    for kk in range(RPT // ZR):
        pltpu.sync_copy(zbuf, acc.at[pl.ds(row0 + kk * ZR, ZR)])

    @pl.when(sid == NS - 1)
    def _ztail():
        pltpu.sync_copy(zbuf.at[pl.ds(0, N - NS * RPT)],
                        acc.at[pl.ds(NS * RPT, N - NS * RPT)])
    plsc.subcore_barrier()

    pltpu.make_async_copy(pk_hbm.at[wid, 0], pk0, psem0).wait()
    pltpu.async_copy(table_hbm.at[pk0.at[0]], rows0, gsem)

    def _chunk(c, b, wait_scatter, guard_pk2, guard_next):
        pk_b, pk_o = pkb[b], pkb[1 - b]
        rows_b, rows_o = rowsb[b], rowsb[1 - b]
        sidx_b, sidx_o = sidxb[b], sidxb[1 - b]
        sval_b = svalb[b]
        psem_o = psemb[1 - b]
        ssem_b, ssem_o = ssemb[b], ssemb[1 - b]

        pltpu.make_async_copy(table_hbm.at[pk_b.at[0]], rows_b, gsem).wait()
        for i in range(CH // 16):
            sidx_b[pl.ds(16 * i, 16)] = pk_b[1, pl.ds(16 * i, 16)]
            sval_b[pl.ds(16 * i, 16)] = plsc.bitcast(
                pk_b[2, pl.ds(16 * i, 16)], _F32)

        def _pk2():
            pltpu.async_copy(pk_hbm.at[wid, c + 2], pk_b, psemb[b])
        if guard_pk2 is None:
            _pk2()
        else:
            pl.when(guard_pk2)(_pk2)

        def _next():
            pltpu.make_async_copy(pk_hbm.at[wid, c + 1], pk_o, psem_o).wait()

            def _ws():
                pltpu.make_async_copy(rows_o, acc.at[sidx_o], ssem_o).wait()
            if wait_scatter:
                _ws()
            pltpu.async_copy(table_hbm.at[pk_o.at[0]], rows_o, gsem)
        if guard_next is None:
            _next()
        else:
            pl.when(guard_next)(_next)

        def _scale(i, carry):
            for u in range(4):
                e = 4 * i + u
                sv = plsc.load_gather(sval_b, [jnp.full((16,), e, jnp.int32)])
                for j in range(D // 16):
                    rows_b[e, pl.ds(j * 16, 16)] = (
                        rows_b[e, pl.ds(j * 16, 16)] * sv)
            return carry

        lax.fori_loop(0, CH // 4, _scale, 0)
        pltpu.async_copy(rows_b, acc.at[sidx_b], ssem_b, add=True)

    _chunk(0, 0, wait_scatter=False, guard_pk2=None, guard_next=None)

    def _pair(t, carry):
        not_last = t < (NCHUNK - 3) // 2
        _chunk(2 * t + 1, 1, wait_scatter=True, guard_pk2=not_last,
               guard_next=None)
        _chunk(2 * t + 2, 0, wait_scatter=True, guard_pk2=not_last,
               guard_next=not_last)
        return carry

    lax.fori_loop(0, (NCHUNK - 1) // 2, _pair, 0)

    pltpu.make_async_copy(rows1, acc.at[sidx1], ssem1).wait()
    pltpu.make_async_copy(rows0, acc.at[sidx0], ssem0).wait()
    plsc.subcore_barrier()

    pltpu.sync_copy(acc.at[pl.ds(row0, RPT)], out_hbm.at[cid, pl.ds(row0, RPT)])

    @pl.when(sid == NS - 1)
    def _wtail():
        pltpu.sync_copy(acc.at[pl.ds(NS * RPT, N - NS * RPT)],
                        out_hbm.at[cid, pl.ds(NS * RPT, N - NS * RPT)])


def _final_body(out0_ref, part_ref, w_out_ref, b_out_ref, y_ref):
    z = jnp.maximum(out0_ref[...] + part_ref[0] + part_ref[1], 0.0)
    y_ref[...] = _DOT_T(z, w_out_ref[...]) + b_out_ref[...]


def _final_call(out0, partials, W_out, b_out):
    return pl.pallas_call(
        _final_body,
        grid=(N // BT,),
        in_specs=[
            pl.BlockSpec((BT, D), lambda i: (i, 0)),
            pl.BlockSpec((NC, BT, D), lambda i: (0, i, 0)),
            pl.BlockSpec((D, D), lambda i: (0, 0)),
            pl.BlockSpec((1, D), lambda i: (0, 0)),
        ],
        out_specs=pl.BlockSpec((BT, D), lambda i: (i, 0)),
        out_shape=jax.ShapeDtypeStruct((N, D), _F32),
    )(out0, partials, W_out, b_out)


def kernel(x, edge_index, adj_values, W_in, b_in, rel_W, W_self, b_self,
           W_out, b_out):
    src = (edge_index[:, 0, :]
           + (jnp.arange(R, dtype=jnp.int32) * N)[:, None]).reshape(
               NW, NCHUNK, CH)
    dst = edge_index[:, 1, :].reshape(NW, NCHUNK, CH)
    val = lax.bitcast_convert_type(adj_values, jnp.int32).reshape(
        NW, NCHUNK, CH)
    packed = jnp.stack([src, dst, val], axis=2)

    W_big = jnp.concatenate(
        [W_self.T, rel_W.transpose(2, 0, 1).reshape(D, R * D)], axis=1)
    out0, table = _tables_call(x, W_in, b_in.reshape(1, D), W_big,
                               b_self.reshape(1, D))
    partials = _sc_agg(table.reshape(R * N, D), packed)
    return _final_call(out0, partials, W_out, b_out.reshape(1, D))


# speedup vs baseline: 5.4901x; 1.2007x over previous
import functools

import jax
import jax.numpy as jnp
from jax import lax
from jax.experimental import pallas as pl
from jax.experimental.pallas import tpu as pltpu
from jax.experimental.pallas import tpu_sc as plsc

N = 10000
D = 128
R = 8
E = 40000

NC = 2
NS = 16
NW = NC * NS
EPT = (R * E) // NW
CH = 80
NCHUNK = EPT // CH
LASTC = NCHUNK - 1
RPT = 624
ZR = 208
BT = 1000

_F32 = jnp.float32
_DOT_T = functools.partial(
    lax.dot_general,
    dimension_numbers=(((1,), (1,)), ((), ())),
    preferred_element_type=_F32,
)
_DOT = functools.partial(
    lax.dot_general,
    dimension_numbers=(((1,), (0,)), ((), ())),
    preferred_element_type=_F32,
)


def _tables_body(x_ref, w_in_ref, b_in_ref, w_big_ref, b_self_ref,
                 out0_ref, table_ref):
    h = jnp.maximum(_DOT_T(x_ref[...], w_in_ref[...]) + b_in_ref[...], 0.0)
    big = _DOT(h, w_big_ref[...])
    out0_ref[...] = big[:, :D] + b_self_ref[...]
    for r in range(R):
        table_ref[r] = big[:, (r + 1) * D:(r + 2) * D]


def _tables_call(x, W_in, b_in, W_big, b_self):
    return pl.pallas_call(
        _tables_body,
        grid=(N // BT,),
        in_specs=[
            pl.BlockSpec((BT, D), lambda i: (i, 0)),
            pl.BlockSpec((D, D), lambda i: (0, 0)),
            pl.BlockSpec((1, D), lambda i: (0, 0)),
            pl.BlockSpec((D, (R + 1) * D), lambda i: (0, 0)),
            pl.BlockSpec((1, D), lambda i: (0, 0)),
        ],
        out_specs=[
            pl.BlockSpec((BT, D), lambda i: (i, 0)),
            pl.BlockSpec((R, BT, D), lambda i: (0, i, 0)),
        ],
        out_shape=[
            jax.ShapeDtypeStruct((N, D), _F32),
            jax.ShapeDtypeStruct((R, N, D), _F32),
        ],
    )(x, W_in, b_in, W_big, b_self)


_MESH = plsc.VectorSubcoreMesh(core_axis_name="c", subcore_axis_name="s")


@functools.partial(
    pl.kernel,
    out_type=jax.ShapeDtypeStruct((NC, N, D), _F32),
    mesh=_MESH,
    scratch_types=[
        pltpu.VMEM_SHARED((N, D), _F32),
        pltpu.VMEM((CH, D), _F32),
        pltpu.VMEM((CH, D), _F32),
        pltpu.VMEM((3, CH), jnp.int32),
        pltpu.VMEM((3, CH), jnp.int32),
        pltpu.VMEM((CH,), jnp.int32),
        pltpu.VMEM((CH,), jnp.int32),
        pltpu.VMEM((CH,), _F32),
        pltpu.VMEM((CH,), _F32),
        pltpu.VMEM((ZR, D), _F32),
        pltpu.SemaphoreType.DMA,
        pltpu.SemaphoreType.DMA,
        pltpu.SemaphoreType.DMA,
        pltpu.SemaphoreType.DMA,
        pltpu.SemaphoreType.DMA,
    ],
    compiler_params=pltpu.CompilerParams(needs_layout_passes=False),
)
def _sc_agg(table_hbm, pk_hbm, out_hbm,
            acc, rows0, rows1, pk0, pk1, sidx0, sidx1, sval0, sval1, zbuf,
            gsem, psem0, psem1, ssem0, ssem1):
    cid = lax.axis_index("c")
    sid = lax.axis_index("s")
    wid = cid * NS + sid
    row0 = sid * RPT

    pkb = (pk0, pk1)
    rowsb = (rows0, rows1)
    sidxb = (sidx0, sidx1)
    svalb = (sval0, sval1)
    psemb = (psem0, psem1)
    ssemb = (ssem0, ssem1)

    pltpu.async_copy(pk_hbm.at[wid, 0], pk0, psem0)
    pltpu.async_copy(pk_hbm.at[wid, 1], pk1, psem1)

    zv = jnp.zeros((16,), _F32)

    def _zrow(i, carry):
        for j in range(D // 16):
            zbuf[i, pl.ds(j * 16, 16)] = zv
        return carry

    lax.fori_loop(0, ZR, _zrow, 0)
    for kk in range(RPT // ZR):
        pltpu.sync_copy(zbuf, acc.at[pl.ds(row0 + kk * ZR, ZR)])

    @pl.when(sid == NS - 1)
    def _ztail():
        pltpu.sync_copy(zbuf.at[pl.ds(0, N - NS * RPT)],
                        acc.at[pl.ds(NS * RPT, N - NS * RPT)])
    plsc.subcore_barrier()

    pltpu.make_async_copy(pk_hbm.at[wid, 0], pk0, psem0).wait()
    pltpu.async_copy(table_hbm.at[pk0.at[0]], rows0, gsem)

    def _chunk(c, b, wait_scatter, guard_pk2, guard_next):
        pk_b, pk_o = pkb[b], pkb[1 - b]
        rows_b, rows_o = rowsb[b], rowsb[1 - b]
        sidx_b, sidx_o = sidxb[b], sidxb[1 - b]
        sval_b = svalb[b]
        psem_o = psemb[1 - b]
        ssem_b, ssem_o = ssemb[b], ssemb[1 - b]

        pltpu.make_async_copy(table_hbm.at[pk_b.at[0]], rows_b, gsem).wait()
        for i in range(CH // 16):
            sidx_b[pl.ds(16 * i, 16)] = pk_b[1, pl.ds(16 * i, 16)]
            sval_b[pl.ds(16 * i, 16)] = plsc.bitcast(
                pk_b[2, pl.ds(16 * i, 16)], _F32)

        def _pk2():
            pltpu.async_copy(pk_hbm.at[wid, c + 2], pk_b, psemb[b])
        if guard_pk2 is None:
            _pk2()
        else:
            pl.when(guard_pk2)(_pk2)

        def _next():
            pltpu.make_async_copy(pk_hbm.at[wid, c + 1], pk_o, psem_o).wait()

            def _ws():
                pltpu.make_async_copy(rows_o, acc.at[sidx_o], ssem_o).wait()
            if wait_scatter:
                _ws()
            pltpu.async_copy(table_hbm.at[pk_o.at[0]], rows_o, gsem)
        if guard_next is None:
            _next()
        else:
            pl.when(guard_next)(_next)

        def _scale(i, carry):
            for u in range(4):
                e = 4 * i + u
                sv = plsc.load_gather(sval_b, [jnp.full((16,), e, jnp.int32)])
                for j in range(D // 16):
                    rows_b[e, pl.ds(j * 16, 16)] = (
                        rows_b[e, pl.ds(j * 16, 16)] * sv)
            return carry

        lax.fori_loop(0, CH // 4, _scale, 0)
        pltpu.async_copy(rows_b, acc.at[sidx_b], ssem_b, add=True)

    _chunk(0, 0, wait_scatter=False, guard_pk2=None, guard_next=None)

    def _pair(t, carry):
        not_last = t < (NCHUNK - 3) // 2
        _chunk(2 * t + 1, 1, wait_scatter=True, guard_pk2=not_last,
               guard_next=None)
        _chunk(2 * t + 2, 0, wait_scatter=True, guard_pk2=not_last,
               guard_next=not_last)
        return carry

    lax.fori_loop(0, (NCHUNK - 1) // 2, _pair, 0)

    pltpu.make_async_copy(rows1, acc.at[sidx1], ssem1).wait()
    pltpu.make_async_copy(rows0, acc.at[sidx0], ssem0).wait()
    plsc.subcore_barrier()

    pltpu.sync_copy(acc.at[pl.ds(row0, RPT)], out_hbm.at[cid, pl.ds(row0, RPT)])

    @pl.when(sid == NS - 1)
    def _wtail():
        pltpu.sync_copy(acc.at[pl.ds(NS * RPT, N - NS * RPT)],
                        out_hbm.at[cid, pl.ds(NS * RPT, N - NS * RPT)])


def _final_body(out0_ref, part_ref, w_out_ref, b_out_ref, y_ref):
    z = jnp.maximum(out0_ref[...] + part_ref[0] + part_ref[1], 0.0)
    y_ref[...] = _DOT_T(z, w_out_ref[...]) + b_out_ref[...]


def _final_call(out0, partials, W_out, b_out):
    return pl.pallas_call(
        _final_body,
        grid=(N // BT,),
        in_specs=[
            pl.BlockSpec((BT, D), lambda i: (i, 0)),
            pl.BlockSpec((NC, BT, D), lambda i: (0, i, 0)),
            pl.BlockSpec((D, D), lambda i: (0, 0)),
            pl.BlockSpec((1, D), lambda i: (0, 0)),
        ],
        out_specs=pl.BlockSpec((BT, D), lambda i: (i, 0)),
        out_shape=jax.ShapeDtypeStruct((N, D), _F32),
    )(out0, partials, W_out, b_out)


def kernel(x, edge_index, adj_values, W_in, b_in, rel_W, W_self, b_self,
           W_out, b_out):
    src = (edge_index[:, 0, :]
           + (jnp.arange(R, dtype=jnp.int32) * N)[:, None]).reshape(
               NW, NCHUNK, CH)
    dst = edge_index[:, 1, :].reshape(NW, NCHUNK, CH)
    val = lax.bitcast_convert_type(adj_values, jnp.int32).reshape(
        NW, NCHUNK, CH)
    packed = jnp.stack([src, dst, val], axis=2)

    W_big = jnp.concatenate(
        [W_self.T, rel_W.transpose(2, 0, 1).reshape(D, R * D)], axis=1)
    out0, table = _tables_call(x, W_in, b_in.reshape(1, D), W_big,
                               b_self.reshape(1, D))
    partials = _sc_agg(table.reshape(R * N, D), packed)
    return _final_call(out0, partials, W_out, b_out.reshape(1, D))
